# Initial kernel scaffold; baseline (speedup 1.0000x reference)
import jax
import jax.numpy as jnp
from jax.experimental import pallas as pl


def kernel(x_src, x_tgt):
    raise NotImplementedError("write your pallas kernel here")


# speedup vs baseline: 28.0626x; 28.0626x over previous
import functools

import numpy as np
import jax
import jax.numpy as jnp
from jax import lax
from jax.experimental import pallas as pl
from jax.experimental.pallas import tpu as pltpu
from jax.experimental.pallas import tpu_sc as plsc

NCORE = 2
NSUB = 16
NW = NCORE * NSUB
LANES = 16

T = 2048
WIN = 2048


def _error_body(x_ref, l_ref, o_ref):
    x = x_ref[0]
    m = jnp.max(x, axis=0, keepdims=True)
    ex = jnp.exp(x - m)
    p = ex / jnp.sum(ex, axis=0, keepdims=True)
    lbl = l_ref[0]
    cls = lax.broadcasted_iota(jnp.int32, (x.shape[0], 1), 0)
    fg = lbl == cls
    o_ref[...] = jnp.where(fg, -(1.0 - p), p)


def _hist_body(e_hbm, out_hbm, ebuf, hista, histf, red, sem):
    wid = lax.axis_index("s") * NCORE + lax.axis_index("c")
    npix_t = e_hbm.shape[1] // NW
    nwin = npix_t // WIN
    num_classes = e_hbm.shape[0]

    zi = jnp.zeros((LANES,), jnp.int32)
    ones = jnp.ones((LANES,), jnp.int32)
    laneoff = jnp.arange(LANES, dtype=jnp.int32) * T

    def _zero(i, _):
        hista[pl.ds(i * LANES, LANES)] = zi
        histf[pl.ds(i * LANES, LANES)] = zi
        return 0

    lax.fori_loop(0, (LANES * T) // LANES, _zero, 0)

    def _class_body(c, _):
        def _win_body(w, _):
            pltpu.sync_copy(
                e_hbm.at[c, pl.ds(wid * npix_t + w * WIN, WIN)], ebuf)

            def _vec_body(j, _):
                v = ebuf[pl.ds(j * LANES, LANES)]
                u = plsc.bitcast(v, jnp.int32)
                fgm = u < 0
                e = jnp.abs(v)
                q = (e * float(T)).astype(jnp.int32)
                q = jnp.minimum(q, T - 1)
                idx = q + laneoff
                plsc.addupdate_scatter(hista, [idx], ones)
                plsc.addupdate_scatter(histf, [idx], ones, mask=fgm)
                return 0

            lax.fori_loop(0, WIN // LANES, _vec_body, 0)
            return 0

        lax.fori_loop(0, nwin, _win_body, 0)

        def _red_body(t, _):
            acc_a = zi
            acc_f = zi
            for l in range(LANES):
                off = l * T + t * LANES
                acc_a = acc_a + hista[pl.ds(off, LANES)]
                acc_f = acc_f + histf[pl.ds(off, LANES)]
                hista[pl.ds(off, LANES)] = zi
                histf[pl.ds(off, LANES)] = zi
            red[pl.ds(t * LANES, LANES)] = acc_a
            red[pl.ds(T + t * LANES, LANES)] = acc_f
            return 0

        lax.fori_loop(0, T // LANES, _red_body, 0)
        pltpu.sync_copy(red, out_hbm.at[c, wid])
        return 0

    lax.fori_loop(0, num_classes, _class_body, 0)


def _make_suffix_mats(num_classes):
    rows = num_classes * NSUB
    j = np.arange(128)
    m128 = (j[:, None] >= j[None, :]).astype(np.float32)
    r = np.arange(rows)
    same = (r[:, None] // NSUB) == (r[None, :] // NSUB)
    mg = (same & (r[None, :] > r[:, None])).astype(np.float32)
    mh = same.astype(np.float32)
    return m128, mg, mh


def _final_body(ha_ref, hf_ref, m128_ref, mg_ref, mh_ref, o_ref):
    ha = jnp.sum(ha_ref[...].astype(jnp.float32), axis=0)
    hf = jnp.sum(hf_ref[...].astype(jnp.float32), axis=0)
    m128 = m128_ref[...]
    mg = mg_ref[...]
    mh = mh_ref[...]
    ra = jnp.sum(ha, axis=1, keepdims=True)
    rf = jnp.sum(hf, axis=1, keepdims=True)
    F = jnp.dot(ha, m128, preferred_element_type=jnp.float32) + \
        jnp.dot(mg, ra, preferred_element_type=jnp.float32)
    S = jnp.dot(hf, m128, preferred_element_type=jnp.float32) + \
        jnp.dot(mg, rf, preferred_element_type=jnp.float32)
    G = jnp.dot(mh, rf, preferred_element_type=jnp.float32)
    den = jnp.maximum(G + F - S, 1.0)
    J = jnp.where(F > 0, 1.0 - (G - S) / den, 0.0)
    num_classes = ha.shape[0] // NSUB
    o_ref[0, 0] = jnp.sum(J) / (float(T) * num_classes)


def kernel(x_src, x_tgt):
    B, C, H, Wd = x_src.shape
    HWp = H * Wd
    N = B * HWp
    CH = 2048
    x3 = x_src.reshape(B, C, HWp)
    l2 = x_tgt.reshape(B, 1, HWp)

    E = pl.pallas_call(
        _error_body,
        grid=(B, HWp // CH),
        in_specs=[
            pl.BlockSpec((1, C, CH), lambda b, k: (b, 0, k)),
            pl.BlockSpec((1, 1, CH), lambda b, k: (b, 0, k)),
        ],
        out_specs=pl.BlockSpec(
            (C, CH), lambda b, k: (0, b * (HWp // CH) + k)),
        out_shape=jax.ShapeDtypeStruct((C, N), jnp.float32),
    )(x3, l2)

    mesh = plsc.VectorSubcoreMesh(core_axis_name="c", subcore_axis_name="s")
    hist = pl.kernel(
        _hist_body,
        out_type=jax.ShapeDtypeStruct((C, NW, 2 * T), jnp.int32),
        mesh=mesh,
        scratch_types=[
            pltpu.VMEM((WIN,), jnp.float32),
            pltpu.VMEM((LANES * T,), jnp.int32),
            pltpu.VMEM((LANES * T,), jnp.int32),
            pltpu.VMEM((2 * T,), jnp.int32),
            pltpu.SemaphoreType.DMA,
        ],
        compiler_params=pltpu.CompilerParams(needs_layout_passes=False),
    )(E)

    h5 = hist.reshape(C, NW, 2, NSUB, 128)
    ha = h5[:, :, 0].transpose(1, 0, 2, 3).reshape(NW, C * NSUB, 128)
    hf = h5[:, :, 1].transpose(1, 0, 2, 3).reshape(NW, C * NSUB, 128)
    m128, mg, mh = _make_suffix_mats(C)
    rows = C * NSUB
    loss = pl.pallas_call(
        _final_body,
        in_specs=[
            pl.BlockSpec((NW, rows, 128), lambda: (0, 0, 0)),
            pl.BlockSpec((NW, rows, 128), lambda: (0, 0, 0)),
            pl.BlockSpec((128, 128), lambda: (0, 0)),
            pl.BlockSpec((rows, rows), lambda: (0, 0)),
            pl.BlockSpec((rows, rows), lambda: (0, 0)),
        ],
        out_specs=pl.BlockSpec(memory_space=pltpu.MemorySpace.SMEM),
        out_shape=jax.ShapeDtypeStruct((1, 1), jnp.float32),
    )(ha, hf, jnp.asarray(m128), jnp.asarray(mg), jnp.asarray(mh))
    return loss[0, 0]


# speedup vs baseline: 33.3439x; 1.1882x over previous
import functools

import numpy as np
import jax
import jax.numpy as jnp
from jax import lax
from jax.experimental import pallas as pl
from jax.experimental.pallas import tpu as pltpu
from jax.experimental.pallas import tpu_sc as plsc

NCORE = 2
NSUB = 16
NW = NCORE * NSUB
LANES = 16

T = 2048
WIN = 4096
UNROLL = 4


def _error_body(x_ref, l_ref, o_ref):
    x = x_ref[0]
    m = jnp.max(x, axis=0, keepdims=True)
    ex = jnp.exp(x - m)
    p = ex / jnp.sum(ex, axis=0, keepdims=True)
    lbl = l_ref[0]
    cls = lax.broadcasted_iota(jnp.int32, (x.shape[0], 1), 0)
    fg = lbl == cls
    o_ref[...] = jnp.where(fg, -(1.0 - p), p)


def _hist_body(e_hbm, out_hbm, ebuf, hista, histf, red, sem0, sem1):
    wid = lax.axis_index("s") * NCORE + lax.axis_index("c")
    npix_t = e_hbm.shape[1] // NW
    nwin = npix_t // WIN
    num_classes = e_hbm.shape[0]
    total = num_classes * nwin

    zi = jnp.zeros((LANES,), jnp.int32)
    ones = jnp.ones((LANES,), jnp.int32)
    laneoff = jnp.arange(LANES, dtype=jnp.int32) * T

    def _copy_g(g, buf):
        c = g // nwin
        w = g - c * nwin
        src = e_hbm.at[c, pl.ds(wid * npix_t + w * WIN, WIN)]
        return pltpu.make_async_copy(
            src, ebuf.at[pl.ds(buf * WIN, WIN)], sem0 if buf == 0 else sem1)

    def _start_g(g):
        @pl.when(g % 2 == 0)
        def _():
            _copy_g(g, 0).start()

        @pl.when(g % 2 == 1)
        def _():
            _copy_g(g, 1).start()

    def _wait_g(g):
        @pl.when(g % 2 == 0)
        def _():
            _copy_g(g, 0).wait()

        @pl.when(g % 2 == 1)
        def _():
            _copy_g(g, 1).wait()

    def _zero(i, _):
        hista[pl.ds(i * LANES, LANES)] = zi
        histf[pl.ds(i * LANES, LANES)] = zi
        return 0

    lax.fori_loop(0, (LANES * T) // LANES, _zero, 0)
    _start_g(0)

    def _class_body(c, _):
        def _win_body(w, _):
            g = c * nwin + w

            @pl.when(g + 1 < total)
            def _():
                _start_g(g + 1)

            _wait_g(g)
            base = (g % 2) * WIN

            def _vec_body(j, _):
                for k in range(UNROLL):
                    v = ebuf[pl.ds(base + (j * UNROLL + k) * LANES, LANES)]
                    u = plsc.bitcast(v, jnp.int32)
                    fgm = u < 0
                    e = jnp.abs(v)
                    q = (e * float(T)).astype(jnp.int32)
                    q = jnp.minimum(q, T - 1)
                    idx = q + laneoff
                    plsc.addupdate_scatter(hista, [idx], ones)
                    plsc.addupdate_scatter(histf, [idx], ones, mask=fgm)
                return 0

            lax.fori_loop(0, WIN // (LANES * UNROLL), _vec_body, 0)
            return 0

        lax.fori_loop(0, nwin, _win_body, 0)

        def _red_body(t, _):
            acc_a = zi
            acc_f = zi
            for l in range(LANES):
                off = l * T + t * LANES
                acc_a = acc_a + hista[pl.ds(off, LANES)]
                acc_f = acc_f + histf[pl.ds(off, LANES)]
                hista[pl.ds(off, LANES)] = zi
                histf[pl.ds(off, LANES)] = zi
            red[pl.ds(t * LANES, LANES)] = acc_a
            red[pl.ds(T + t * LANES, LANES)] = acc_f
            return 0

        lax.fori_loop(0, T // LANES, _red_body, 0)
        pltpu.sync_copy(red, out_hbm.at[c, wid])
        return 0

    lax.fori_loop(0, num_classes, _class_body, 0)


def _make_suffix_mats(num_classes):
    rows = num_classes * NSUB
    j = np.arange(128)
    m128 = (j[:, None] >= j[None, :]).astype(np.float32)
    r = np.arange(rows)
    same = (r[:, None] // NSUB) == (r[None, :] // NSUB)
    mg = (same & (r[None, :] > r[:, None])).astype(np.float32)
    mh = same.astype(np.float32)
    return m128, mg, mh


def _final_body(ha_ref, hf_ref, m128_ref, mg_ref, mh_ref, o_ref):
    ha = jnp.sum(ha_ref[...].astype(jnp.float32), axis=0)
    hf = jnp.sum(hf_ref[...].astype(jnp.float32), axis=0)
    m128 = m128_ref[...]
    mg = mg_ref[...]
    mh = mh_ref[...]
    ra = jnp.sum(ha, axis=1, keepdims=True)
    rf = jnp.sum(hf, axis=1, keepdims=True)
    F = jnp.dot(ha, m128, preferred_element_type=jnp.float32) + \
        jnp.dot(mg, ra, preferred_element_type=jnp.float32)
    S = jnp.dot(hf, m128, preferred_element_type=jnp.float32) + \
        jnp.dot(mg, rf, preferred_element_type=jnp.float32)
    G = jnp.dot(mh, rf, preferred_element_type=jnp.float32)
    den = jnp.maximum(G + F - S, 1.0)
    J = jnp.where(F > 0, 1.0 - (G - S) / den, 0.0)
    num_classes = ha.shape[0] // NSUB
    o_ref[0, 0] = jnp.sum(J) / (float(T) * num_classes)


def kernel(x_src, x_tgt):
    B, C, H, Wd = x_src.shape
    HWp = H * Wd
    N = B * HWp
    CH = 2048
    x3 = x_src.reshape(B, C, HWp)
    l2 = x_tgt.reshape(B, 1, HWp)

    E = pl.pallas_call(
        _error_body,
        grid=(B, HWp // CH),
        in_specs=[
            pl.BlockSpec((1, C, CH), lambda b, k: (b, 0, k)),
            pl.BlockSpec((1, 1, CH), lambda b, k: (b, 0, k)),
        ],
        out_specs=pl.BlockSpec(
            (C, CH), lambda b, k: (0, b * (HWp // CH) + k)),
        out_shape=jax.ShapeDtypeStruct((C, N), jnp.float32),
    )(x3, l2)

    mesh = plsc.VectorSubcoreMesh(core_axis_name="c", subcore_axis_name="s")
    hist = pl.kernel(
        _hist_body,
        out_type=jax.ShapeDtypeStruct((C, NW, 2 * T), jnp.int32),
        mesh=mesh,
        scratch_types=[
            pltpu.VMEM((2 * WIN,), jnp.float32),
            pltpu.VMEM((LANES * T,), jnp.int32),
            pltpu.VMEM((LANES * T,), jnp.int32),
            pltpu.VMEM((2 * T,), jnp.int32),
            pltpu.SemaphoreType.DMA,
            pltpu.SemaphoreType.DMA,
        ],
        compiler_params=pltpu.CompilerParams(needs_layout_passes=False),
    )(E)

    h5 = hist.reshape(C, NW, 2, NSUB, 128)
    ha = h5[:, :, 0].transpose(1, 0, 2, 3).reshape(NW, C * NSUB, 128)
    hf = h5[:, :, 1].transpose(1, 0, 2, 3).reshape(NW, C * NSUB, 128)
    m128, mg, mh = _make_suffix_mats(C)
    rows = C * NSUB
    loss = pl.pallas_call(
        _final_body,
        in_specs=[
            pl.BlockSpec((NW, rows, 128), lambda: (0, 0, 0)),
            pl.BlockSpec((NW, rows, 128), lambda: (0, 0, 0)),
            pl.BlockSpec((128, 128), lambda: (0, 0)),
            pl.BlockSpec((rows, rows), lambda: (0, 0)),
            pl.BlockSpec((rows, rows), lambda: (0, 0)),
        ],
        out_specs=pl.BlockSpec(memory_space=pltpu.MemorySpace.SMEM),
        out_shape=jax.ShapeDtypeStruct((1, 1), jnp.float32),
    )(ha, hf, jnp.asarray(m128), jnp.asarray(mg), jnp.asarray(mh))
    return loss[0, 0]


# speedup vs baseline: 33.7552x; 1.0123x over previous
import functools

import numpy as np
import jax
import jax.numpy as jnp
from jax import lax
from jax.experimental import pallas as pl
from jax.experimental.pallas import tpu as pltpu
from jax.experimental.pallas import tpu_sc as plsc

NCORE = 2
NSUB = 16
NW = NCORE * NSUB
LANES = 16

T = 2048
WIN = 4096
UNROLL = 4


def _error_body(x_ref, l_ref, o_ref):
    x = x_ref[0]
    m = jnp.max(x, axis=0, keepdims=True)
    ex = jnp.exp(x - m)
    p = ex / jnp.sum(ex, axis=0, keepdims=True)
    lbl = l_ref[0]
    cls = lax.broadcasted_iota(jnp.int32, (x.shape[0], 1), 0)
    fg = lbl == cls
    o_ref[...] = jnp.where(fg, -(1.0 - p), p)


def _hist_body(e_hbm, out_hbm, ebuf, hist0, hist1, red, sem0, sem1):
    wid = lax.axis_index("s") * NCORE + lax.axis_index("c")
    npix_t = e_hbm.shape[1] // NW
    nwin = npix_t // WIN
    num_classes = e_hbm.shape[0]
    total = num_classes * nwin

    zi = jnp.zeros((LANES,), jnp.int32)
    laneoff = jnp.arange(LANES, dtype=jnp.int32) * T
    one = jnp.full((LANES,), 1, jnp.uint32)

    def _copy_g(g, buf):
        c = g // nwin
        w = g - c * nwin
        src = e_hbm.at[c, pl.ds(wid * npix_t + w * WIN, WIN)]
        return pltpu.make_async_copy(
            src, ebuf.at[pl.ds(buf * WIN, WIN)], sem0 if buf == 0 else sem1)

    def _start_g(g):
        @pl.when(g % 2 == 0)
        def _():
            _copy_g(g, 0).start()

        @pl.when(g % 2 == 1)
        def _():
            _copy_g(g, 1).start()

    def _wait_g(g):
        @pl.when(g % 2 == 0)
        def _():
            _copy_g(g, 0).wait()

        @pl.when(g % 2 == 1)
        def _():
            _copy_g(g, 1).wait()

    def _zero(i, _):
        hist0[pl.ds(i * LANES, LANES)] = zi
        hist1[pl.ds(i * LANES, LANES)] = zi
        return 0

    lax.fori_loop(0, (LANES * T) // LANES, _zero, 0)
    _start_g(0)

    def _class_body(c, _):
        def _win_body(w, _):
            g = c * nwin + w

            @pl.when(g + 1 < total)
            def _():
                _start_g(g + 1)

            _wait_g(g)
            base = (g % 2) * WIN

            def _vec_body(j, _):
                for k in range(UNROLL):
                    v = ebuf[pl.ds(base + (j * UNROLL + k) * LANES, LANES)]
                    u = plsc.bitcast(v, jnp.uint32)
                    upd = plsc.bitcast(((u >> 31) << 16) + one, jnp.int32)
                    e = jnp.abs(v)
                    q = (e * float(T)).astype(jnp.int32)
                    q = jnp.minimum(q, T - 1)
                    idx = q + laneoff
                    plsc.addupdate_scatter(
                        hist0 if k % 2 == 0 else hist1, [idx], upd)
                return 0

            lax.fori_loop(0, WIN // (LANES * UNROLL), _vec_body, 0)
            return 0

        lax.fori_loop(0, nwin, _win_body, 0)

        def _red_body(t, _):
            acc = zi
            for l in range(LANES):
                off = l * T + t * LANES
                acc = acc + hist0[pl.ds(off, LANES)] + hist1[pl.ds(off, LANES)]
                hist0[pl.ds(off, LANES)] = zi
                hist1[pl.ds(off, LANES)] = zi
            red[pl.ds(t * LANES, LANES)] = acc
            return 0

        lax.fori_loop(0, T // LANES, _red_body, 0)
        pltpu.sync_copy(red, out_hbm.at[c, wid])
        return 0

    lax.fori_loop(0, num_classes, _class_body, 0)


def _final_body(h_ref, m_ref, o_ref):
    hu = lax.bitcast_convert_type(h_ref[...], jnp.uint32)
    ha3 = (hu & jnp.uint32(0xFFFF)).astype(jnp.float32)
    hf3 = (hu >> jnp.uint32(16)).astype(jnp.float32)
    ha = jnp.sum(ha3, axis=1)
    hf = jnp.sum(hf3, axis=1)
    m = m_ref[...]
    F = jnp.dot(ha, m, preferred_element_type=jnp.float32)
    S = jnp.dot(hf, m, preferred_element_type=jnp.float32)
    G = jnp.sum(hf, axis=1, keepdims=True)
    den = jnp.maximum(G + F - S, 1.0)
    J = jnp.where(F > 0, 1.0 - (G - S) / den, 0.0)
    o_ref[0, 0] = jnp.sum(J) / (float(T) * h_ref.shape[0])


_SUFFIX_M = np.tril(np.ones((T, T), np.float32))


def kernel(x_src, x_tgt):
    B, C, H, Wd = x_src.shape
    HWp = H * Wd
    N = B * HWp
    CH = 2048
    x3 = x_src.reshape(B, C, HWp)
    l2 = x_tgt.reshape(B, 1, HWp)

    E = pl.pallas_call(
        _error_body,
        grid=(B, HWp // CH),
        in_specs=[
            pl.BlockSpec((1, C, CH), lambda b, k: (b, 0, k)),
            pl.BlockSpec((1, 1, CH), lambda b, k: (b, 0, k)),
        ],
        out_specs=pl.BlockSpec(
            (C, CH), lambda b, k: (0, b * (HWp // CH) + k)),
        out_shape=jax.ShapeDtypeStruct((C, N), jnp.float32),
    )(x3, l2)

    mesh = plsc.VectorSubcoreMesh(core_axis_name="c", subcore_axis_name="s")
    hist = pl.kernel(
        _hist_body,
        out_type=jax.ShapeDtypeStruct((C, NW, T), jnp.int32),
        mesh=mesh,
        scratch_types=[
            pltpu.VMEM((2 * WIN,), jnp.float32),
            pltpu.VMEM((LANES * T,), jnp.int32),
            pltpu.VMEM((LANES * T,), jnp.int32),
            pltpu.VMEM((T,), jnp.int32),
            pltpu.SemaphoreType.DMA,
            pltpu.SemaphoreType.DMA,
        ],
        compiler_params=pltpu.CompilerParams(needs_layout_passes=False),
    )(E)

    loss = pl.pallas_call(
        _final_body,
        in_specs=[
            pl.BlockSpec((C, NW, T), lambda: (0, 0, 0)),
            pl.BlockSpec((T, T), lambda: (0, 0)),
        ],
        out_specs=pl.BlockSpec(memory_space=pltpu.MemorySpace.SMEM),
        out_shape=jax.ShapeDtypeStruct((1, 1), jnp.float32),
    )(hist, jnp.asarray(_SUFFIX_M))
    return loss[0, 0]
